# baseline (speedup 1.0000x reference)
import jax
import jax.numpy as jnp
from jax.experimental import pallas as pl
from jax.experimental.pallas import tpu as pltpu

KVAL = 32
B = 128
D = 2048
H = 16384
TH = 512
NE = H // TH
TD = 128
ND = D // TD
NG = 128
NBUF = 3


def _start_wd_copy(wd_hbm, wd_buf, sems, tile):
    slot = jax.lax.rem(tile, NBUF)
    pltpu.make_async_copy(
        wd_hbm.at[pl.ds(tile * TD, TD), :],
        wd_buf.at[slot],
        sems.at[slot],
    ).start()


def _fused(x_ref, we_ref, be_ref, wd_hbm, sparse_ref, recon_ref,
           feat_ref, sfb_ref, gmax_ref, thr_ref, wd_buf, sems):
    i = pl.program_id(0)

    @pl.when(i < NE)
    def _encode():
        xb = x_ref[...].astype(jnp.bfloat16)
        wb = we_ref[...].astype(jnp.bfloat16)
        f = jax.lax.dot_general(xb, wb, (((1,), (1,)), ((), ())),
                                preferred_element_type=jnp.float32)
        f = jnp.maximum(f + be_ref[:, pl.ds(i * TH, TH)], 0.0)
        feat_ref[:, pl.ds(i * TH, TH)] = f
        tile_gm = jnp.max(f.reshape(B, TH // NG, NG), axis=1)

        @pl.when(i == 0)
        def _():
            gmax_ref[...] = tile_gm

        @pl.when(i > 0)
        def _():
            gmax_ref[...] = jnp.maximum(gmax_ref[...], tile_gm)

    @pl.when(i == NE - 1)
    def _threshold():
        for t in range(NBUF):
            _start_wd_copy(wd_hbm, wd_buf, sems, t)

        gm = gmax_ref[...]
        rowmax = jnp.max(gm, axis=1, keepdims=True)
        hi0 = jax.lax.bitcast_convert_type(rowmax, jnp.int32) + 1
        lo0 = jnp.zeros_like(hi0)

        def cnt_gm(t):
            return jnp.sum((gm >= t).astype(jnp.float32), axis=1, keepdims=True)

        def gm_body(_, carry):
            l, h = carry
            mid = l + (h - l) // 2
            ok = cnt_gm(jax.lax.bitcast_convert_type(mid, jnp.float32)) >= KVAL
            return jnp.where(ok, mid, l), jnp.where(ok, h, mid)

        lo1, _ = jax.lax.fori_loop(0, 31, gm_body, (lo0, hi0))

        feats = feat_ref[...]

        def cnt_full(t):
            return jnp.sum((feats >= t).astype(jnp.float32), axis=1,
                           keepdims=True)

        def step(carry):
            l, h, c = carry
            mid = l + (h - l) // 2
            cm = cnt_full(jax.lax.bitcast_convert_type(mid, jnp.float32))
            ok = cm >= KVAL
            return (jnp.where(ok, mid, l), jnp.where(ok, h, mid),
                    jnp.where(ok, cm, c))

        def w_cond(carry):
            l, h, c = carry
            return jnp.any((h - l > 1) & (c != KVAL))

        def w_body(carry):
            return step(step(carry))

        cnt1 = cnt_full(jax.lax.bitcast_convert_type(lo1, jnp.float32))
        lo, _, _ = jax.lax.while_loop(w_cond, w_body, (lo1, hi0, cnt1))
        thr = jax.lax.bitcast_convert_type(lo, jnp.float32)
        thr_ref[...] = thr
        sfb_ref[...] = jnp.where(feats >= thr, feats, 0.0).astype(jnp.bfloat16)

    @pl.when(i >= NE)
    def _decode():
        j = i - NE
        feats = feat_ref[:, pl.ds(j * (H // ND), H // ND)]
        sparse_ref[...] = jnp.where(feats >= thr_ref[...], feats, 0.0)
        slot = jax.lax.rem(j, NBUF)
        pltpu.make_async_copy(
            wd_hbm.at[pl.ds(j * TD, TD), :], wd_buf.at[slot], sems.at[slot],
        ).wait()
        wd = wd_buf[slot].astype(jnp.bfloat16)
        recon_ref[...] = jax.lax.dot_general(
            sfb_ref[...], wd, (((1,), (1,)), ((), ())),
            preferred_element_type=jnp.float32)

        @pl.when(j + NBUF < ND)
        def _():
            _start_wd_copy(wd_hbm, wd_buf, sems, j + NBUF)


def kernel(x, W_enc, b_enc, W_dec):
    sparse, recon = pl.pallas_call(
        _fused,
        grid=(NE + ND,),
        in_specs=[
            pl.BlockSpec((B, D), lambda i: (0, 0)),
            pl.BlockSpec((TH, D), lambda i: (jnp.minimum(i, NE - 1), 0)),
            pl.BlockSpec((1, H), lambda i: (0, 0)),
            pl.BlockSpec(memory_space=pltpu.MemorySpace.HBM),
        ],
        out_specs=[
            pl.BlockSpec((B, H // ND), lambda i: (0, jnp.maximum(i - NE, 0))),
            pl.BlockSpec((B, TD), lambda i: (0, jnp.maximum(i - NE, 0))),
        ],
        out_shape=[
            jax.ShapeDtypeStruct((B, H), jnp.float32),
            jax.ShapeDtypeStruct((B, D), jnp.float32),
        ],
        scratch_shapes=[
            pltpu.VMEM((B, H), jnp.float32),
            pltpu.VMEM((B, H), jnp.bfloat16),
            pltpu.VMEM((B, NG), jnp.float32),
            pltpu.VMEM((B, 1), jnp.float32),
            pltpu.VMEM((NBUF, TD, H), jnp.float32),
            pltpu.SemaphoreType.DMA((NBUF,)),
        ],
        compiler_params=pltpu.CompilerParams(
            dimension_semantics=("arbitrary",),
        ),
    )(x, W_enc, b_enc.reshape(1, H), W_dec)
    return (sparse, recon)


# speedup vs baseline: 1.0756x; 1.0756x over previous
import jax
import jax.numpy as jnp
from jax.experimental import pallas as pl
from jax.experimental.pallas import tpu as pltpu

KVAL = 32
B = 128
D = 2048
H = 16384
TH = 1024
NE = H // TH
TD = 128
ND = D // TD
NG = 128
NBUF = 2


def _start_wd_copy(wd_hbm, wd_buf, sems, tile):
    slot = jax.lax.rem(tile, NBUF)
    pltpu.make_async_copy(
        wd_hbm.at[pl.ds(tile * TD, TD), :],
        wd_buf.at[slot],
        sems.at[slot],
    ).start()


def _fused(x_ref, we_ref, be_ref, wd_hbm, sparse_ref, recon_ref,
           feat_ref, sfb_ref, gmax_ref, thr_ref, wd_buf, sems):
    i = pl.program_id(0)

    @pl.when(i < NE)
    def _encode():
        xb = x_ref[...].astype(jnp.bfloat16)
        wb = we_ref[...].astype(jnp.bfloat16)
        f = jax.lax.dot_general(xb, wb, (((1,), (1,)), ((), ())),
                                preferred_element_type=jnp.float32)
        f = jnp.maximum(f + be_ref[:, pl.ds(i * TH, TH)], 0.0)
        feat_ref[:, pl.ds(i * TH, TH)] = f
        tile_gm = jnp.max(f.reshape(B, TH // NG, NG), axis=1)

        @pl.when(i == 0)
        def _():
            gmax_ref[...] = tile_gm

        @pl.when(i > 0)
        def _():
            gmax_ref[...] = jnp.maximum(gmax_ref[...], tile_gm)

    @pl.when(i == NE - 1)
    def _threshold():
        for t in range(NBUF):
            _start_wd_copy(wd_hbm, wd_buf, sems, t)

        gm = gmax_ref[...]
        rowmax = jnp.max(gm, axis=1, keepdims=True)
        hi0 = jax.lax.bitcast_convert_type(rowmax, jnp.int32) + 1
        lo0 = jnp.zeros_like(hi0)

        def cnt_gm(t):
            return jnp.sum((gm >= t).astype(jnp.float32), axis=1, keepdims=True)

        def gm_body(_, carry):
            l, h = carry
            mid = l + (h - l) // 2
            ok = cnt_gm(jax.lax.bitcast_convert_type(mid, jnp.float32)) >= KVAL
            return jnp.where(ok, mid, l), jnp.where(ok, h, mid)

        lo1, _ = jax.lax.fori_loop(0, 31, gm_body, (lo0, hi0))

        feats = feat_ref[...]

        def cnt_full(t):
            return jnp.sum((feats >= t).astype(jnp.float32), axis=1,
                           keepdims=True)

        def step(carry):
            l, h, c = carry
            mid = l + (h - l) // 2
            cm = cnt_full(jax.lax.bitcast_convert_type(mid, jnp.float32))
            ok = cm >= KVAL
            return (jnp.where(ok, mid, l), jnp.where(ok, h, mid),
                    jnp.where(ok, cm, c))

        def w_cond(carry):
            l, h, c = carry
            return jnp.any((h - l > 1) & (c != KVAL))

        def w_body(carry):
            return step(step(carry))

        cnt1 = cnt_full(jax.lax.bitcast_convert_type(lo1, jnp.float32))
        lo, _, _ = jax.lax.while_loop(w_cond, w_body, (lo1, hi0, cnt1))
        thr = jax.lax.bitcast_convert_type(lo, jnp.float32)
        thr_ref[...] = thr
        sfb_ref[...] = jnp.where(feats >= thr, feats, 0.0).astype(jnp.bfloat16)

    @pl.when(i >= NE)
    def _decode():
        j = i - NE
        feats = feat_ref[:, pl.ds(j * (H // ND), H // ND)]
        sparse_ref[...] = jnp.where(feats >= thr_ref[...], feats, 0.0)
        slot = jax.lax.rem(j, NBUF)
        pltpu.make_async_copy(
            wd_hbm.at[pl.ds(j * TD, TD), :], wd_buf.at[slot], sems.at[slot],
        ).wait()
        wd = wd_buf[slot].astype(jnp.bfloat16)
        recon_ref[...] = jax.lax.dot_general(
            sfb_ref[...], wd, (((1,), (1,)), ((), ())),
            preferred_element_type=jnp.float32)

        @pl.when(j + NBUF < ND)
        def _():
            _start_wd_copy(wd_hbm, wd_buf, sems, j + NBUF)


def kernel(x, W_enc, b_enc, W_dec):
    sparse, recon = pl.pallas_call(
        _fused,
        grid=(NE + ND,),
        in_specs=[
            pl.BlockSpec((B, D), lambda i: (0, 0)),
            pl.BlockSpec((TH, D), lambda i: (jnp.minimum(i, NE - 1), 0)),
            pl.BlockSpec((1, H), lambda i: (0, 0)),
            pl.BlockSpec(memory_space=pltpu.MemorySpace.HBM),
        ],
        out_specs=[
            pl.BlockSpec((B, H // ND), lambda i: (0, jnp.maximum(i - NE, 0))),
            pl.BlockSpec((B, TD), lambda i: (0, jnp.maximum(i - NE, 0))),
        ],
        out_shape=[
            jax.ShapeDtypeStruct((B, H), jnp.float32),
            jax.ShapeDtypeStruct((B, D), jnp.float32),
        ],
        scratch_shapes=[
            pltpu.VMEM((B, H), jnp.float32),
            pltpu.VMEM((B, H), jnp.bfloat16),
            pltpu.VMEM((B, NG), jnp.float32),
            pltpu.VMEM((B, 1), jnp.float32),
            pltpu.VMEM((NBUF, TD, H), jnp.float32),
            pltpu.SemaphoreType.DMA((NBUF,)),
        ],
        compiler_params=pltpu.CompilerParams(
            dimension_semantics=("arbitrary",),
        ),
    )(x, W_enc, b_enc.reshape(1, H), W_dec)
    return (sparse, recon)


# speedup vs baseline: 1.0815x; 1.0055x over previous
import jax
import jax.numpy as jnp
from jax.experimental import pallas as pl
from jax.experimental.pallas import tpu as pltpu

KVAL = 32
B = 128
D = 2048
H = 16384
TH = 1024
NE = H // TH
TD = 128
ND = D // TD
NG = 128
NBUF = 2


def _start_wd_copy(wd_hbm, wd_buf, sems, tile):
    slot = jax.lax.rem(tile, NBUF)
    pltpu.make_async_copy(
        wd_hbm.at[pl.ds(tile * TD, TD), :],
        wd_buf.at[slot],
        sems.at[slot],
    ).start()


def _fused(x_ref, we_ref, be_ref, wd_hbm, sparse_ref, recon_ref,
           feat_ref, sfb_ref, gmax_ref, thr_ref, wd_buf, sems):
    i = pl.program_id(0)

    @pl.when(i < NE)
    def _encode():
        xb = x_ref[...].astype(jnp.bfloat16)
        wb = we_ref[...].astype(jnp.bfloat16)
        f = jax.lax.dot_general(xb, wb, (((1,), (1,)), ((), ())),
                                preferred_element_type=jnp.float32)
        f = jnp.maximum(f + be_ref[:, pl.ds(i * TH, TH)], 0.0)
        feat_ref[:, pl.ds(i * TH, TH)] = f
        tile_gm = jnp.max(f.reshape(B, TH // NG, NG), axis=1)

        @pl.when(i == 0)
        def _():
            gmax_ref[...] = tile_gm

        @pl.when(i > 0)
        def _():
            gmax_ref[...] = jnp.maximum(gmax_ref[...], tile_gm)

    @pl.when(i == NE - 2)
    def _prefetch_wd():
        for t in range(NBUF):
            _start_wd_copy(wd_hbm, wd_buf, sems, t)

    @pl.when(i == NE - 1)
    def _threshold():
        gm = gmax_ref[...]
        rowmax = jnp.max(gm, axis=1, keepdims=True)
        hi0 = jax.lax.bitcast_convert_type(rowmax, jnp.int32) + 1
        lo0 = jnp.zeros_like(hi0)

        def cnt_gm(t):
            return jnp.sum((gm >= t).astype(jnp.float32), axis=1, keepdims=True)

        def gm_body(_, carry):
            l, h = carry
            mid = l + (h - l) // 2
            ok = cnt_gm(jax.lax.bitcast_convert_type(mid, jnp.float32)) >= KVAL
            return jnp.where(ok, mid, l), jnp.where(ok, h, mid)

        lo1, _ = jax.lax.fori_loop(0, 31, gm_body, (lo0, hi0))

        feats = feat_ref[...]

        def cnt_full(t):
            return jnp.sum((feats >= t).astype(jnp.float32), axis=1,
                           keepdims=True)

        def step(carry):
            l, h, c = carry
            mid = l + (h - l) // 2
            cm = cnt_full(jax.lax.bitcast_convert_type(mid, jnp.float32))
            ok = cm >= KVAL
            return (jnp.where(ok, mid, l), jnp.where(ok, h, mid),
                    jnp.where(ok, cm, c))

        def w_cond(carry):
            l, h, c = carry
            return jnp.any((h - l > 1) & (c != KVAL))

        def w_body(carry):
            return step(step(carry))

        cnt1 = cnt_full(jax.lax.bitcast_convert_type(lo1, jnp.float32))
        carry0 = step(step(step(step((lo1, hi0, cnt1)))))
        lo, _, _ = jax.lax.while_loop(w_cond, w_body, carry0)
        thr = jax.lax.bitcast_convert_type(lo, jnp.float32)
        thr_ref[...] = thr
        sfb_ref[...] = jnp.where(feats >= thr, feats, 0.0).astype(jnp.bfloat16)

    @pl.when(i >= NE)
    def _decode():
        j = i - NE
        feats = feat_ref[:, pl.ds(j * (H // ND), H // ND)]
        sparse_ref[...] = jnp.where(feats >= thr_ref[...], feats, 0.0)
        slot = jax.lax.rem(j, NBUF)
        pltpu.make_async_copy(
            wd_hbm.at[pl.ds(j * TD, TD), :], wd_buf.at[slot], sems.at[slot],
        ).wait()
        wd = wd_buf[slot].astype(jnp.bfloat16)
        recon_ref[...] = jax.lax.dot_general(
            sfb_ref[...], wd, (((1,), (1,)), ((), ())),
            preferred_element_type=jnp.float32)

        @pl.when(j + NBUF < ND)
        def _():
            _start_wd_copy(wd_hbm, wd_buf, sems, j + NBUF)


def kernel(x, W_enc, b_enc, W_dec):
    sparse, recon = pl.pallas_call(
        _fused,
        grid=(NE + ND,),
        in_specs=[
            pl.BlockSpec((B, D), lambda i: (0, 0)),
            pl.BlockSpec((TH, D), lambda i: (jnp.minimum(i, NE - 1), 0)),
            pl.BlockSpec((1, H), lambda i: (0, 0)),
            pl.BlockSpec(memory_space=pltpu.MemorySpace.HBM),
        ],
        out_specs=[
            pl.BlockSpec((B, H // ND), lambda i: (0, jnp.maximum(i - NE, 0))),
            pl.BlockSpec((B, TD), lambda i: (0, jnp.maximum(i - NE, 0))),
        ],
        out_shape=[
            jax.ShapeDtypeStruct((B, H), jnp.float32),
            jax.ShapeDtypeStruct((B, D), jnp.float32),
        ],
        scratch_shapes=[
            pltpu.VMEM((B, H), jnp.float32),
            pltpu.VMEM((B, H), jnp.bfloat16),
            pltpu.VMEM((B, NG), jnp.float32),
            pltpu.VMEM((B, 1), jnp.float32),
            pltpu.VMEM((NBUF, TD, H), jnp.float32),
            pltpu.SemaphoreType.DMA((NBUF,)),
        ],
        compiler_params=pltpu.CompilerParams(
            dimension_semantics=("arbitrary",),
        ),
    )(x, W_enc, b_enc.reshape(1, H), W_dec)
    return (sparse, recon)
